# baseline (speedup 1.0000x reference)
import functools

import jax
import jax.numpy as jnp
from jax import lax
from jax.experimental import pallas as pl
from jax.experimental.pallas import tpu as pltpu
from jax.experimental.pallas import tpu_sc as plsc

BATCH = 4096
EMBED_DIM = 64


def _make_sc_gather(B, D):
    info = plsc.get_sparse_core_info()
    NC, NS = info.num_cores, info.num_subcores
    NW = NC * NS
    assert B % (8 * NW) == 0
    b_per_w = B // NW
    mesh = plsc.VectorSubcoreMesh(core_axis_name="c", subcore_axis_name="s")

    @functools.partial(
        pl.kernel,
        mesh=mesh,
        out_type=[
            jax.ShapeDtypeStruct((B, D), jnp.float32),
            jax.ShapeDtypeStruct((B, D), jnp.float32),
        ],
        scratch_types=[
            pltpu.VMEM((b_per_w,), jnp.int32),
            pltpu.VMEM((b_per_w,), jnp.int32),
            pltpu.VMEM((b_per_w, D), jnp.float32),
            pltpu.VMEM((b_per_w, D), jnp.float32),
            pltpu.SemaphoreType.DMA,
        ],
        compiler_params=pltpu.CompilerParams(use_tc_tiling_on_sc=False),
    )
    def gather_k(uid_hbm, mid_hbm, ut_hbm, mt_hbm, uout_hbm, mout_hbm,
                 uidx_v, midx_v, urows_v, mrows_v, sem):
        wid = lax.axis_index("s") * NC + lax.axis_index("c")
        base = wid * b_per_w
        pltpu.sync_copy(uid_hbm.at[pl.ds(base, b_per_w)], uidx_v)
        pltpu.sync_copy(mid_hbm.at[pl.ds(base, b_per_w)], midx_v)
        cu = pltpu.async_copy(ut_hbm.at[uidx_v], urows_v, sem)
        cm = pltpu.async_copy(mt_hbm.at[midx_v], mrows_v, sem)
        cu.wait()
        cm.wait()
        pltpu.sync_copy(urows_v, uout_hbm.at[pl.ds(base, b_per_w)])
        pltpu.sync_copy(mrows_v, mout_hbm.at[pl.ds(base, b_per_w)])

    return gather_k


def _mlp_body(u_ref, m_ref, w1a_ref, w1b_ref, b1_ref, w2_ref, b2_ref,
              w3_ref, b3_ref, o_ref):
    h1 = jnp.dot(u_ref[...], w1a_ref[...], preferred_element_type=jnp.float32)
    h1 += jnp.dot(m_ref[...], w1b_ref[...], preferred_element_type=jnp.float32)
    h1 = jnp.maximum(h1 + b1_ref[...], 0.0)
    h2 = jnp.dot(h1, w2_ref[...], preferred_element_type=jnp.float32)
    h2 = jnp.maximum(h2 + b2_ref[...], 0.0)
    o_ref[...] = (
        jnp.dot(h2, w3_ref[...], preferred_element_type=jnp.float32)
        + b3_ref[...]
    )


def _make_mlp(B, D, BB):
    grid = (B // BB,)
    const = lambda i: (0, 0)
    return pl.pallas_call(
        _mlp_body,
        grid=grid,
        in_specs=[
            pl.BlockSpec((BB, D), lambda i: (i, 0)),
            pl.BlockSpec((BB, D), lambda i: (i, 0)),
            pl.BlockSpec((D, 256), const),
            pl.BlockSpec((D, 256), const),
            pl.BlockSpec((1, 256), const),
            pl.BlockSpec((256, 64), const),
            pl.BlockSpec((1, 64), const),
            pl.BlockSpec((64, 1), const),
            pl.BlockSpec((1, 1), const),
        ],
        out_specs=pl.BlockSpec((BB, 1), lambda i: (i, 0)),
        out_shape=jax.ShapeDtypeStruct((B, 1), jnp.float32),
    )


@jax.jit
def kernel(user_id, movie_id, user_table, movie_table, W1, b1, W2, b2, W3, b3):
    B = user_id.shape[0]
    D = user_table.shape[1]
    gather_k = _make_sc_gather(B, D)
    u_emb, m_emb = gather_k(
        user_id.astype(jnp.int32), movie_id.astype(jnp.int32),
        user_table, movie_table)
    mlp = _make_mlp(B, D, 1024)
    return mlp(
        u_emb, m_emb,
        W1[:D], W1[D:],
        b1.reshape(1, 256),
        W2, b2.reshape(1, 64),
        W3, b3.reshape(1, 1),
    )
